# baseline (speedup 1.0000x reference)
import functools

import jax
import jax.numpy as jnp
from jax import lax
from jax.experimental import pallas as pl
from jax.experimental.pallas import tpu as pltpu
from jax.experimental.pallas import tpu_sc as plsc

_D = 32
_B = 16384 * 26
_NW = 32
_BPW = _B // _NW
_C = 256
_NCHUNK = _BPW // _C
_NBUF = 8
_PRE = 6


def _build():
    mesh = plsc.VectorSubcoreMesh(core_axis_name="c", subcore_axis_name="s")

    @functools.partial(
        pl.kernel,
        mesh=mesh,
        compiler_params=pltpu.CompilerParams(use_tc_tiling_on_sc=False),
        out_type=jax.ShapeDtypeStruct((_B, _D), jnp.float32),
        scratch_types=[
            pltpu.VMEM((_BPW,), jnp.int32),
            pltpu.VMEM((_NBUF, _C, _D), jnp.float32),
            pltpu.SemaphoreType.DMA((_NBUF,)),
            pltpu.SemaphoreType.DMA((_NBUF,)),
        ],
    )
    def k(idx_hbm, table_hbm, out_hbm, idx_v, rows_v, g_sem, s_sem):
        wid = lax.axis_index("s") * 2 + lax.axis_index("c")
        base = wid * _BPW
        pltpu.sync_copy(idx_hbm.at[pl.ds(base, _BPW)], idx_v)

        def gather(c, buf):
            return pltpu.async_copy(
                table_hbm.at[idx_v.at[pl.ds(c * _C, _C)]],
                rows_v.at[buf], g_sem.at[buf])

        def store(c, buf):
            return pltpu.async_copy(
                rows_v.at[buf], out_hbm.at[pl.ds(base + c * _C, _C)],
                s_sem.at[buf])

        g = [None] * _NCHUNK
        s = [None] * _NCHUNK
        for c in range(_PRE):
            g[c] = gather(c, c % _NBUF)
        for c in range(_NCHUNK):
            g[c].wait()
            s[c] = store(c, c % _NBUF)
            nxt = c + _PRE
            if nxt < _NCHUNK:
                old = nxt - _NBUF
                if old >= 0:
                    s[old].wait()
                g[nxt] = gather(nxt, nxt % _NBUF)
        for c in range(max(0, _NCHUNK - _NBUF), _NCHUNK):
            s[c].wait()

    return k


_gather_call = _build()


@jax.jit
def kernel(x, table):
    idx = x.reshape(-1)
    out = _gather_call(idx, table)
    return out.reshape(x.shape + (table.shape[1],))
